# baseline (speedup 1.0000x reference)
import functools

import jax
import jax.numpy as jnp
from jax import lax
from jax.experimental import pallas as pl
from jax.experimental.pallas import tpu as pltpu
from jax.experimental.pallas import tpu_sc as plsc

N_NODES = 10000
N_EDGES = 320000
D_IN = 128
D_HID = 256
D_OUT = 128
N_HYPER = 10000

NC = 2
NS = 16
NW = NC * NS
EPW = N_EDGES // NW
NPAD = 10240
RPT = NPAD // NS
CW = 8


def _fill_zero_2d(ref, rows, cols, dtype=jnp.float32):
    lanes = 32 if dtype == jnp.bfloat16 else 16
    per_row = cols // lanes

    def body(t, c):
        ref[t // per_row, pl.ds((t % per_row) * lanes, lanes)] = jnp.zeros(
            (lanes,), dtype)
        return c

    lax.fori_loop(0, rows * per_row, body, 0)


def _make_sc_pass(with_counts):
    if with_counts:
        CH, NFULL, TAIL, ZR = 80, 125, 0, 32
    else:
        CH, NFULL, TAIL, ZR = 128, 78, 16, 64
    NPAIR = NFULL // 2
    LEFTOVER = NFULL % 2
    assert NPAIR >= 2

    out_type = [jax.ShapeDtypeStruct((NC, NPAD, 128), jnp.bfloat16)]
    scratch = [
        pltpu.VMEM_SHARED((NPAD, 128), jnp.bfloat16),
        pltpu.VMEM((CH,), jnp.int32),
        pltpu.VMEM((CH,), jnp.int32),
        pltpu.VMEM((CH,), jnp.int32),
        pltpu.VMEM((CH,), jnp.int32),
        pltpu.VMEM((CH,), jnp.int32),
        pltpu.VMEM((CH,), jnp.int32),
        pltpu.VMEM((CH, 128), jnp.bfloat16),
        pltpu.VMEM((CH, 128), jnp.bfloat16),
        pltpu.VMEM((ZR, 128), jnp.bfloat16),
        pltpu.SemaphoreType.DMA,
        pltpu.SemaphoreType.DMA,
        pltpu.SemaphoreType.DMA,
        pltpu.SemaphoreType.DMA,
        pltpu.SemaphoreType.DMA,
        pltpu.SemaphoreType.DMA,
    ]
    if TAIL:
        scratch += [
            pltpu.VMEM((TAIL,), jnp.int32),
            pltpu.VMEM((TAIL,), jnp.int32),
            pltpu.VMEM((TAIL, 128), jnp.bfloat16),
        ]
    if with_counts:
        out_type += [
            jax.ShapeDtypeStruct((NC, NPAD, CW), jnp.float32),
            jax.ShapeDtypeStruct((NC, NPAD, CW), jnp.float32),
        ]
        scratch += [
            pltpu.VMEM_SHARED((NPAD, CW), jnp.float32),
            pltpu.VMEM_SHARED((NPAD, CW), jnp.float32),
            pltpu.VMEM((ZR, CW), jnp.float32),
            pltpu.VMEM((CH, CW), jnp.float32),
            pltpu.VMEM((CH,), jnp.int32),
            pltpu.VMEM((CH,), jnp.int32),
            pltpu.SemaphoreType.DMA,
            pltpu.SemaphoreType.DMA,
        ]

    mesh = plsc.VectorSubcoreMesh(core_axis_name="c", subcore_axis_name="s")

    def body(src_hbm, isrc_hbm, idst_hbm, *rest):
        it = iter(rest)
        if with_counts:
            ones_hbm = next(it)
            zer_hbm = next(it)
        out_hbm = next(it)
        if with_counts:
            dcnt_hbm = next(it)
            bcnt_hbm = next(it)
        acc = next(it)
        isrc0, idst0, isrc1, idst1 = next(it), next(it), next(it), next(it)
        idsts0, idsts1 = next(it), next(it)
        rows0, rows1 = next(it), next(it)
        zbuf = next(it)
        sem_i0, sem_i1, sem_g0, sem_g1, sem_s0, sem_s1 = (
            next(it), next(it), next(it), next(it), next(it), next(it))
        if TAIL:
            isrc_t, idst_t, rows_t = next(it), next(it), next(it)
        if with_counts:
            dacc, bacc, cbuf, ones_v, isrcs0, isrcs1, sem_c0, sem_c1 = (
                next(it), next(it), next(it), next(it), next(it), next(it),
                next(it), next(it))
            isrcs = (isrcs0, isrcs1)
            sem_c = (sem_c0, sem_c1)

        cid = lax.axis_index("c")
        sid = lax.axis_index("s")
        wid = cid * NS + sid
        ebase = wid * EPW
        rbase = sid * RPT

        bufs = ((isrc0, idst0, idsts0, rows0, sem_i0, sem_g0, sem_s0),
                (isrc1, idst1, idsts1, rows1, sem_i1, sem_g1, sem_s1))

        def issue_idx(j, p):
            isb, idb = bufs[p][0], bufs[p][1]
            base = ebase + j * CH
            pltpu.async_copy(isrc_hbm.at[pl.ds(base, CH)], isb, sem_i := bufs[p][4])
            pltpu.async_copy(idst_hbm.at[pl.ds(base, CH)], idb, sem_i)

        def wait_idx(p):
            isb, idb, sem_i = bufs[p][0], bufs[p][1], bufs[p][4]
            pltpu.make_async_copy(isrc_hbm.at[pl.ds(0, CH)], isb, sem_i).wait()
            pltpu.make_async_copy(idst_hbm.at[pl.ds(0, CH)], idb, sem_i).wait()

        def issue_gather(p):
            isb, rb, sem_g = bufs[p][0], bufs[p][3], bufs[p][5]
            pltpu.async_copy(src_hbm.at[isb], rb, sem_g)

        def wait_gather(p):
            isb, rb, sem_g = bufs[p][0], bufs[p][3], bufs[p][5]
            pltpu.make_async_copy(src_hbm.at[isb], rb, sem_g).wait()

        def shadow_and_issue(p):
            isb, idb, ids, rb, _, _, sem_s = bufs[p]
            for t in range(CH // 16):
                ids[pl.ds(t * 16, 16)] = idb[pl.ds(t * 16, 16)]
            if with_counts:
                iss = isrcs[p]
                for t in range(CH // 16):
                    iss[pl.ds(t * 16, 16)] = isb[pl.ds(t * 16, 16)]
                pltpu.async_copy(ones_v, dacc.at[iss], sem_c[p], add=True)
                pltpu.async_copy(ones_v, bacc.at[ids], sem_c[p], add=True)
            pltpu.async_copy(rb, acc.at[ids], sem_s, add=True)

        def wait_scatter(p):
            ids, rb, sem_s = bufs[p][2], bufs[p][3], bufs[p][6]
            pltpu.make_async_copy(rb, acc.at[ids], sem_s).wait()
            if with_counts:
                iss = isrcs[p]
                pltpu.make_async_copy(ones_v, dacc.at[iss], sem_c[p]).wait()
                pltpu.make_async_copy(ones_v, bacc.at[ids], sem_c[p]).wait()

        issue_idx(0, 0)
        issue_idx(1, 1)

        _fill_zero_2d(zbuf, ZR, 128, jnp.bfloat16)
        for s in range(RPT // ZR):
            pltpu.sync_copy(zbuf, acc.at[pl.ds(rbase + s * ZR, ZR)])
        if with_counts:
            pltpu.sync_copy(zer_hbm, cbuf)
            for s in range(RPT // ZR):
                pltpu.sync_copy(cbuf, dacc.at[pl.ds(rbase + s * ZR, ZR)])
                pltpu.sync_copy(cbuf, bacc.at[pl.ds(rbase + s * ZR, ZR)])
            pltpu.sync_copy(ones_hbm, ones_v)
        plsc.subcore_barrier()

        for p in (0, 1):
            wait_idx(p)
            issue_gather(p)
        for p in (0, 1):
            wait_gather(p)
            shadow_and_issue(p)
            issue_idx(p + 2, p)

        def pair_body(k, c):
            for p in (0, 1):
                wait_scatter(p)
                wait_idx(p)
                issue_gather(p)
            for p in (0, 1):
                j = 2 * k + p
                wait_gather(p)
                shadow_and_issue(p)

                @pl.when(j + 2 < NFULL)
                def _():
                    issue_idx(j + 2, p)
            return c

        lax.fori_loop(1, NPAIR, pair_body, 0)

        if LEFTOVER:
            wait_scatter(0)
            wait_idx(0)
            issue_gather(0)
            wait_gather(0)
            shadow_and_issue(0)
            wait_scatter(0)
            wait_scatter(1)
        else:
            wait_scatter(0)
            wait_scatter(1)

        if TAIL:
            base = ebase + NFULL * CH
            pltpu.sync_copy(isrc_hbm.at[pl.ds(base, TAIL)], isrc_t)
            pltpu.sync_copy(idst_hbm.at[pl.ds(base, TAIL)], idst_t)
            pltpu.async_copy(src_hbm.at[isrc_t], rows_t, sem_g0).wait()
            pltpu.sync_copy(rows_t, acc.at[idst_t], add=True)
        plsc.subcore_barrier()

        for s in range(RPT // ZR):
            r0 = rbase + s * ZR
            pltpu.sync_copy(acc.at[pl.ds(r0, ZR)], zbuf)
            pltpu.sync_copy(zbuf, out_hbm.at[cid, pl.ds(r0, ZR)])
        if with_counts:
            for s in range(RPT // ZR):
                r0 = rbase + s * ZR
                pltpu.sync_copy(dacc.at[pl.ds(r0, ZR)], cbuf)
                pltpu.sync_copy(cbuf, dcnt_hbm.at[cid, pl.ds(r0, ZR)])
                pltpu.sync_copy(bacc.at[pl.ds(r0, ZR)], cbuf)
                pltpu.sync_copy(cbuf, bcnt_hbm.at[cid, pl.ds(r0, ZR)])

    return pl.kernel(
        body,
        out_type=out_type if with_counts else out_type[0],
        mesh=mesh,
        scratch_types=scratch,
        compiler_params=pltpu.CompilerParams(use_tc_tiling_on_sc=False),
    )


_sc_pass_counts = _make_sc_pass(True)
_sc_pass = _make_sc_pass(False)


def _make_sc_layer_fused():
    CH, NFULL, TAIL, ZR = 128, 156, 32, 64
    NPAIR = NFULL // 2
    EPT = N_EDGES // NS

    out_type = [
        jax.ShapeDtypeStruct((NC, NPAD, 64), jnp.bfloat16),
        jax.ShapeDtypeStruct((NC, NPAD, 64), jnp.bfloat16),
    ]
    scratch = [
        pltpu.VMEM_SHARED((NPAD, 64), jnp.bfloat16),
        pltpu.VMEM_SHARED((NPAD, 64), jnp.bfloat16),
        pltpu.VMEM((CH,), jnp.int32),
        pltpu.VMEM((CH,), jnp.int32),
        pltpu.VMEM((CH,), jnp.int32),
        pltpu.VMEM((CH,), jnp.int32),
        pltpu.VMEM((CH,), jnp.int32),
        pltpu.VMEM((CH,), jnp.int32),
        pltpu.VMEM((CH, 64), jnp.bfloat16),
        pltpu.VMEM((CH, 64), jnp.bfloat16),
        pltpu.VMEM((ZR, 64), jnp.bfloat16),
        pltpu.VMEM((RPT, 64), jnp.bfloat16),
        pltpu.VMEM((RPT, 32), jnp.bfloat16),
        pltpu.SemaphoreType.DMA,
        pltpu.SemaphoreType.DMA,
        pltpu.SemaphoreType.DMA,
        pltpu.SemaphoreType.DMA,
        pltpu.SemaphoreType.DMA,
        pltpu.SemaphoreType.DMA,
        pltpu.VMEM((TAIL,), jnp.int32),
        pltpu.VMEM((TAIL,), jnp.int32),
        pltpu.VMEM((TAIL, 64), jnp.bfloat16),
    ]

    mesh = plsc.VectorSubcoreMesh(core_axis_name="c", subcore_axis_name="s")

    def body(src_hbm, isrc_hbm, idst_hbm, bcnt_hbm, r_hbm, q_hbm, *rest):
        (eacc, nacc, g0, s0b, g1, s1b, sh0, sh1, rows0, rows1, zbuf, qbuf,
         cntb, sem_i0, sem_i1, sem_g0, sem_g1, sem_s0, sem_s1,
         gt, st, rt) = rest

        cid = lax.axis_index("c")
        sid = lax.axis_index("s")
        ebase = sid * EPT
        rbase = sid * RPT

        bufs = ((g0, s0b, sh0, rows0, sem_i0, sem_g0, sem_s0),
                (g1, s1b, sh1, rows1, sem_i1, sem_g1, sem_s1))

        def edge_phase(src3, gat_hbm, sct_hbm, acc):
            def issue_idx(j, p):
                gb, sb, sem_i = bufs[p][0], bufs[p][1], bufs[p][4]
                base = ebase + j * CH
                pltpu.async_copy(gat_hbm.at[pl.ds(base, CH)], gb, sem_i)
                pltpu.async_copy(sct_hbm.at[pl.ds(base, CH)], sb, sem_i)

            def wait_idx(p):
                gb, sb, sem_i = bufs[p][0], bufs[p][1], bufs[p][4]
                pltpu.make_async_copy(gat_hbm.at[pl.ds(0, CH)], gb, sem_i).wait()
                pltpu.make_async_copy(sct_hbm.at[pl.ds(0, CH)], sb, sem_i).wait()

            def issue_gather(p):
                gb, rb, sem_g = bufs[p][0], bufs[p][3], bufs[p][5]
                pltpu.async_copy(src3.at[cid].at[gb], rb, sem_g)

            def wait_gather(p):
                gb, rb, sem_g = bufs[p][0], bufs[p][3], bufs[p][5]
                pltpu.make_async_copy(src3.at[cid].at[gb], rb, sem_g).wait()

            def shadow_and_issue(p):
                sb, sh, rb, sem_s = bufs[p][1], bufs[p][2], bufs[p][3], bufs[p][6]
                for t in range(CH // 16):
                    sh[pl.ds(t * 16, 16)] = sb[pl.ds(t * 16, 16)]
                pltpu.async_copy(rb, acc.at[sh], sem_s, add=True)

            def wait_scatter(p):
                sh, rb, sem_s = bufs[p][2], bufs[p][3], bufs[p][6]
                pltpu.make_async_copy(rb, acc.at[sh], sem_s).wait()

            issue_idx(0, 0)
            issue_idx(1, 1)
            for p in (0, 1):
                wait_idx(p)
                issue_gather(p)
            for p in (0, 1):
                wait_gather(p)
                shadow_and_issue(p)
                issue_idx(p + 2, p)

            def pair_body(k, c):
                for p in (0, 1):
                    wait_scatter(p)
                    wait_idx(p)
                    issue_gather(p)
                for p in (0, 1):
                    j = 2 * k + p
                    wait_gather(p)
                    shadow_and_issue(p)

                    @pl.when(j + 2 < NFULL)
                    def _():
                        issue_idx(j + 2, p)
                return c

            lax.fori_loop(1, NPAIR, pair_body, 0)
            wait_scatter(0)
            wait_scatter(1)

            base = ebase + NFULL * CH
            pltpu.sync_copy(gat_hbm.at[pl.ds(base, TAIL)], gt)
            pltpu.sync_copy(sct_hbm.at[pl.ds(base, TAIL)], st)
            pltpu.async_copy(src3.at[cid].at[gt], rt, sem_g0).wait()
            pltpu.sync_copy(rt, acc.at[st], add=True)

        _fill_zero_2d(zbuf, ZR, 64, jnp.bfloat16)
        for s in range(RPT // ZR):
            pltpu.sync_copy(zbuf, eacc.at[pl.ds(rbase + s * ZR, ZR)])
            pltpu.sync_copy(zbuf, nacc.at[pl.ds(rbase + s * ZR, ZR)])
        plsc.subcore_barrier()

        edge_phase(src_hbm, isrc_hbm, idst_hbm, eacc)
        plsc.subcore_barrier()

        pltpu.sync_copy(bcnt_hbm.at[pl.ds(rbase, RPT)], cntb)
        pltpu.sync_copy(eacc.at[pl.ds(rbase, RPT)], qbuf)

        one_b = jnp.ones((32,), jnp.bfloat16)
        zero_b = jnp.zeros((32,), jnp.bfloat16)

        def srow(r, c):
            cnt = cntb[r, pl.ds(0, 32)]
            sc = jnp.where(cnt > 0, one_b / jnp.maximum(cnt, one_b), zero_b)
            for g in range(2):
                v = qbuf[r, pl.ds(g * 32, 32)]
                qbuf[r, pl.ds(g * 32, 32)] = v * sc
            return c

        lax.fori_loop(0, RPT, srow, 0)
        pltpu.sync_copy(qbuf, q_hbm.at[cid, pl.ds(rbase, RPT)])
        plsc.subcore_barrier()

        edge_phase(q_hbm, idst_hbm, isrc_hbm, nacc)
        plsc.subcore_barrier()

        for s in range(RPT // ZR):
            r0 = rbase + s * ZR
            pltpu.sync_copy(nacc.at[pl.ds(r0, ZR)], zbuf)
            pltpu.sync_copy(zbuf, r_hbm.at[cid, pl.ds(r0, ZR)])

    return pl.kernel(
        body,
        out_type=out_type,
        mesh=mesh,
        scratch_types=scratch,
        compiler_params=pltpu.CompilerParams(use_tc_tiling_on_sc=False),
    )


_sc_layer_fused = _make_sc_layer_fused()


_ROWS_BLK = 2000
_GRID = N_NODES // _ROWS_BLK


def _inv_counts(c_ref):
    cnt = c_ref[0, :, 0:1] + c_ref[1, :, 0:1]
    return jnp.where(cnt > 0, 1.0 / jnp.maximum(cnt, 1.0), 0.0)


def _scale_body(p_ref, c_ref, q_ref, bb_ref):
    binv = _inv_counts(c_ref)
    p = p_ref[0].astype(jnp.float32) + p_ref[1].astype(jnp.float32)
    q_ref[...] = (p * binv).astype(jnp.bfloat16)
    cnt = c_ref[0, :, 0:1] + c_ref[1, :, 0:1]
    bb_ref[...] = jnp.broadcast_to(cnt, (cnt.shape[0], 32)).astype(jnp.bfloat16)


def _tc_scale(p, cnt):
    return pl.pallas_call(
        _scale_body,
        grid=(_GRID,),
        in_specs=[
            pl.BlockSpec((NC, _ROWS_BLK, 128), lambda i: (0, i, 0)),
            pl.BlockSpec((NC, _ROWS_BLK, CW), lambda i: (0, i, 0)),
        ],
        out_specs=[pl.BlockSpec((_ROWS_BLK, 128), lambda i: (i, 0)),
                   pl.BlockSpec((_ROWS_BLK, 32), lambda i: (i, 0))],
        out_shape=[jax.ShapeDtypeStruct((N_NODES, 128), jnp.bfloat16),
                   jax.ShapeDtypeStruct((NPAD, 32), jnp.bfloat16)],
    )(p, cnt)


def _mm_body(r_ref, c_ref, w1_ref, b1_ref, w2_ref, o_ref):
    dinv = _inv_counts(c_ref)
    s = (r_ref[0].astype(jnp.float32) + r_ref[1].astype(jnp.float32)) * dinv
    h = lax.dot_general(s, w1_ref[...], (((1,), (1,)), ((), ())),
                        preferred_element_type=jnp.float32)
    h = jnp.maximum(h + b1_ref[...], 0.0)
    o = lax.dot_general(h, w2_ref[...], (((1,), (1,)), ((), ())),
                        preferred_element_type=jnp.float32
                        ).astype(jnp.bfloat16)
    o_ref[0, :, :] = o[:, :64]
    o_ref[1, :, :] = o[:, 64:]


def _tc_mm(r, cnt, W1, b1, W2):
    return pl.pallas_call(
        _mm_body,
        grid=(_GRID,),
        in_specs=[
            pl.BlockSpec((NC, _ROWS_BLK, 128), lambda i: (0, i, 0)),
            pl.BlockSpec((NC, _ROWS_BLK, CW), lambda i: (0, i, 0)),
            pl.BlockSpec((D_HID, D_IN), lambda i: (0, 0)),
            pl.BlockSpec((1, D_HID), lambda i: (0, 0)),
            pl.BlockSpec((D_OUT, D_HID), lambda i: (0, 0)),
        ],
        out_specs=pl.BlockSpec((NC, _ROWS_BLK, 64), lambda i: (0, i, 0)),
        out_shape=jax.ShapeDtypeStruct((NC, N_NODES, 64), jnp.bfloat16),
    )(r, cnt, W1, b1, W2)


def _final_body(r_ref, c_ref, b2_ref, o_ref):
    dinv = _inv_counts(c_ref)
    r = jnp.concatenate([r_ref[0], r_ref[1]], axis=1).astype(jnp.float32)
    o_ref[...] = r * dinv + b2_ref[...]


def _tc_final(r, cnt, b2):
    return pl.pallas_call(
        _final_body,
        grid=(_GRID,),
        in_specs=[
            pl.BlockSpec((NC, _ROWS_BLK, 64), lambda i: (0, i, 0)),
            pl.BlockSpec((NC, _ROWS_BLK, CW), lambda i: (0, i, 0)),
            pl.BlockSpec((1, D_OUT), lambda i: (0, 0)),
        ],
        out_specs=pl.BlockSpec((_ROWS_BLK, D_OUT), lambda i: (i, 0)),
        out_shape=jax.ShapeDtypeStruct((N_NODES, D_OUT), jnp.float32),
    )(r, cnt, b2)


@jax.jit
def kernel(x, edge_index, W1, b1, W2, b2):
    node_idx = edge_index[0]
    hyper_idx = edge_index[1]

    ones_pat = jnp.zeros((80, CW), jnp.float32).at[:, 0].set(1.0)
    zer_pat = jnp.zeros((32, CW), jnp.float32)
    p1, dc, bc = _sc_pass_counts(x.astype(jnp.bfloat16), node_idx, hyper_idx,
                                 ones_pat, zer_pat)
    q1, bc_bf = _tc_scale(p1, bc)
    r1 = _sc_pass(q1, hyper_idx, node_idx)
    xw2 = _tc_mm(r1, dc, W1, b1.reshape(1, D_HID), W2)
    r2, _ = _sc_layer_fused(xw2, node_idx, hyper_idx, bc_bf)
    return _tc_final(r2, dc, b2.reshape(1, D_OUT))


# speedup vs baseline: 1.0786x; 1.0786x over previous
import functools

import jax
import jax.numpy as jnp
from jax import lax
from jax.experimental import pallas as pl
from jax.experimental.pallas import tpu as pltpu
from jax.experimental.pallas import tpu_sc as plsc

N_NODES = 10000
N_EDGES = 320000
D_IN = 128
D_HID = 256
D_OUT = 128
N_HYPER = 10000

NC = 2
NS = 16
NW = NC * NS
EPW = N_EDGES // NW
NPAD = 10240
RPT = NPAD // NS
CW = 16


def _fill_zero_2d(ref, rows, cols, dtype=jnp.float32):
    lanes = 32 if dtype == jnp.bfloat16 else 16
    per_row = cols // lanes

    def body(t, c):
        ref[t // per_row, pl.ds((t % per_row) * lanes, lanes)] = jnp.zeros(
            (lanes,), dtype)
        return c

    lax.fori_loop(0, rows * per_row, body, 0)


def _make_sc_pass(with_counts):
    if with_counts:
        CH, NFULL, TAIL, ZR = 128, 78, 16, 64
    else:
        CH, NFULL, TAIL, ZR = 128, 78, 16, 64
    NPAIR = NFULL // 2
    LEFTOVER = NFULL % 2
    assert NPAIR >= 2

    out_type = [jax.ShapeDtypeStruct((NC, NPAD, 128), jnp.bfloat16)]
    scratch = [
        pltpu.VMEM_SHARED((NPAD, 128), jnp.bfloat16),
        pltpu.VMEM((CH,), jnp.int32),
        pltpu.VMEM((CH,), jnp.int32),
        pltpu.VMEM((CH,), jnp.int32),
        pltpu.VMEM((CH,), jnp.int32),
        pltpu.VMEM((CH,), jnp.int32),
        pltpu.VMEM((CH,), jnp.int32),
        pltpu.VMEM((CH, 128), jnp.bfloat16),
        pltpu.VMEM((CH, 128), jnp.bfloat16),
        pltpu.VMEM((ZR, 128), jnp.bfloat16),
        pltpu.SemaphoreType.DMA,
        pltpu.SemaphoreType.DMA,
        pltpu.SemaphoreType.DMA,
        pltpu.SemaphoreType.DMA,
        pltpu.SemaphoreType.DMA,
        pltpu.SemaphoreType.DMA,
    ]
    if TAIL:
        scratch += [
            pltpu.VMEM((TAIL,), jnp.int32),
            pltpu.VMEM((TAIL,), jnp.int32),
            pltpu.VMEM((TAIL, 128), jnp.bfloat16),
        ]
    if with_counts:
        out_type += [
            jax.ShapeDtypeStruct((NC, NPAD, CW), jnp.float32),
            jax.ShapeDtypeStruct((NC, NPAD, CW), jnp.float32),
        ]
        scratch += [
            pltpu.VMEM_SHARED((NPAD, CW), jnp.float32),
            pltpu.VMEM_SHARED((NPAD, CW), jnp.float32),
            pltpu.VMEM((ZR, CW), jnp.float32),
            pltpu.VMEM((CH, CW), jnp.float32),
            pltpu.VMEM((CH,), jnp.int32),
            pltpu.VMEM((CH,), jnp.int32),
            pltpu.SemaphoreType.DMA,
            pltpu.SemaphoreType.DMA,
            pltpu.VMEM((16, CW), jnp.float32),
        ]

    mesh = plsc.VectorSubcoreMesh(core_axis_name="c", subcore_axis_name="s")

    def body(src_hbm, isrc_hbm, idst_hbm, out_hbm, *rest):
        it = iter(rest)
        if with_counts:
            dcnt_hbm = next(it)
            bcnt_hbm = next(it)
        acc = next(it)
        isrc0, idst0, isrc1, idst1 = next(it), next(it), next(it), next(it)
        idsts0, idsts1 = next(it), next(it)
        rows0, rows1 = next(it), next(it)
        zbuf = next(it)
        sem_i0, sem_i1, sem_g0, sem_g1, sem_s0, sem_s1 = (
            next(it), next(it), next(it), next(it), next(it), next(it))
        if TAIL:
            isrc_t, idst_t, rows_t = next(it), next(it), next(it)
        if with_counts:
            (dacc, bacc, cbuf, ones_v, isrcs0, isrcs1, sem_c0, sem_c1,
             ones_t) = (next(it), next(it), next(it), next(it), next(it),
                        next(it), next(it), next(it), next(it))
            isrcs = (isrcs0, isrcs1)
            sem_c = (sem_c0, sem_c1)

        cid = lax.axis_index("c")
        sid = lax.axis_index("s")
        wid = cid * NS + sid
        ebase = wid * EPW
        rbase = sid * RPT

        bufs = ((isrc0, idst0, idsts0, rows0, sem_i0, sem_g0, sem_s0),
                (isrc1, idst1, idsts1, rows1, sem_i1, sem_g1, sem_s1))

        def issue_idx(j, p):
            isb, idb = bufs[p][0], bufs[p][1]
            base = ebase + j * CH
            pltpu.async_copy(isrc_hbm.at[pl.ds(base, CH)], isb, sem_i := bufs[p][4])
            pltpu.async_copy(idst_hbm.at[pl.ds(base, CH)], idb, sem_i)

        def wait_idx(p):
            isb, idb, sem_i = bufs[p][0], bufs[p][1], bufs[p][4]
            pltpu.make_async_copy(isrc_hbm.at[pl.ds(0, CH)], isb, sem_i).wait()
            pltpu.make_async_copy(idst_hbm.at[pl.ds(0, CH)], idb, sem_i).wait()

        def issue_gather(p):
            isb, rb, sem_g = bufs[p][0], bufs[p][3], bufs[p][5]
            pltpu.async_copy(src_hbm.at[isb], rb, sem_g)

        def wait_gather(p):
            isb, rb, sem_g = bufs[p][0], bufs[p][3], bufs[p][5]
            pltpu.make_async_copy(src_hbm.at[isb], rb, sem_g).wait()

        def shadow_and_issue(p):
            isb, idb, ids, rb, _, _, sem_s = bufs[p]
            for t in range(CH // 16):
                ids[pl.ds(t * 16, 16)] = idb[pl.ds(t * 16, 16)]
            if with_counts:
                iss = isrcs[p]
                for t in range(CH // 16):
                    iss[pl.ds(t * 16, 16)] = isb[pl.ds(t * 16, 16)]
                pltpu.async_copy(ones_v, dacc.at[iss], sem_c[p], add=True)
                pltpu.async_copy(ones_v, bacc.at[ids], sem_c[p], add=True)
            pltpu.async_copy(rb, acc.at[ids], sem_s, add=True)

        def wait_scatter(p):
            ids, rb, sem_s = bufs[p][2], bufs[p][3], bufs[p][6]
            pltpu.make_async_copy(rb, acc.at[ids], sem_s).wait()
            if with_counts:
                iss = isrcs[p]
                pltpu.make_async_copy(ones_v, dacc.at[iss], sem_c[p]).wait()
                pltpu.make_async_copy(ones_v, bacc.at[ids], sem_c[p]).wait()

        issue_idx(0, 0)
        issue_idx(1, 1)

        _fill_zero_2d(zbuf, ZR, 128, jnp.bfloat16)
        for s in range(RPT // ZR):
            pltpu.sync_copy(zbuf, acc.at[pl.ds(rbase + s * ZR, ZR)])
        if with_counts:
            _fill_zero_2d(cbuf, ZR, CW)
            for s in range(RPT // ZR):
                pltpu.sync_copy(cbuf, dacc.at[pl.ds(rbase + s * ZR, ZR)])
                pltpu.sync_copy(cbuf, bacc.at[pl.ds(rbase + s * ZR, ZR)])

            def fill_ones(t, c):
                ones_v[t, :] = jnp.where(
                    lax.iota(jnp.int32, 16) == 0, 1.0, 0.0
                ).astype(jnp.float32)
                return c

            lax.fori_loop(0, CH, fill_ones, 0)

            def fill_ones_t(t, c):
                ones_t[t, :] = jnp.where(
                    lax.iota(jnp.int32, 16) == 0, 1.0, 0.0
                ).astype(jnp.float32)
                return c

            lax.fori_loop(0, TAIL, fill_ones_t, 0)
        plsc.subcore_barrier()

        for p in (0, 1):
            wait_idx(p)
            issue_gather(p)
        for p in (0, 1):
            wait_gather(p)
            shadow_and_issue(p)
            issue_idx(p + 2, p)

        def pair_body(k, c):
            for p in (0, 1):
                wait_scatter(p)
                wait_idx(p)
                issue_gather(p)
            for p in (0, 1):
                j = 2 * k + p
                wait_gather(p)
                shadow_and_issue(p)

                @pl.when(j + 2 < NFULL)
                def _():
                    issue_idx(j + 2, p)
            return c

        lax.fori_loop(1, NPAIR, pair_body, 0)

        if LEFTOVER:
            wait_scatter(0)
            wait_idx(0)
            issue_gather(0)
            wait_gather(0)
            shadow_and_issue(0)
            wait_scatter(0)
            wait_scatter(1)
        else:
            wait_scatter(0)
            wait_scatter(1)

        if TAIL:
            base = ebase + NFULL * CH
            pltpu.sync_copy(isrc_hbm.at[pl.ds(base, TAIL)], isrc_t)
            pltpu.sync_copy(idst_hbm.at[pl.ds(base, TAIL)], idst_t)
            pltpu.async_copy(src_hbm.at[isrc_t], rows_t, sem_g0).wait()
            pltpu.sync_copy(rows_t, acc.at[idst_t], add=True)
            if with_counts:
                pltpu.sync_copy(ones_t, dacc.at[isrc_t], add=True)
                pltpu.sync_copy(ones_t, bacc.at[idst_t], add=True)
        plsc.subcore_barrier()

        for s in range(RPT // ZR):
            r0 = rbase + s * ZR
            pltpu.sync_copy(acc.at[pl.ds(r0, ZR)], zbuf)
            pltpu.sync_copy(zbuf, out_hbm.at[cid, pl.ds(r0, ZR)])
        if with_counts:
            for s in range(RPT // ZR):
                r0 = rbase + s * ZR
                pltpu.sync_copy(dacc.at[pl.ds(r0, ZR)], cbuf)
                pltpu.sync_copy(cbuf, dcnt_hbm.at[cid, pl.ds(r0, ZR)])
                pltpu.sync_copy(bacc.at[pl.ds(r0, ZR)], cbuf)
                pltpu.sync_copy(cbuf, bcnt_hbm.at[cid, pl.ds(r0, ZR)])

    return pl.kernel(
        body,
        out_type=out_type if with_counts else out_type[0],
        mesh=mesh,
        scratch_types=scratch,
        compiler_params=pltpu.CompilerParams(use_tc_tiling_on_sc=False),
    )


_sc_pass_counts = _make_sc_pass(True)
_sc_pass = _make_sc_pass(False)


_ROWS_BLK = 2000
_GRID = N_NODES // _ROWS_BLK


def _inv_counts(c_ref):
    cnt = c_ref[0] + c_ref[1]
    return jnp.where(cnt > 0, 1.0 / jnp.maximum(cnt, 1.0), 0.0)


def _scale_body(p_ref, c_ref, q_ref):
    binv = _inv_counts(c_ref)
    p = p_ref[0].astype(jnp.float32) + p_ref[1].astype(jnp.float32)
    q_ref[...] = (p * binv).astype(jnp.bfloat16)


def _tc_scale(p, cnt):
    return pl.pallas_call(
        _scale_body,
        grid=(_GRID,),
        in_specs=[
            pl.BlockSpec((NC, _ROWS_BLK, 128), lambda i: (0, i, 0)),
            pl.BlockSpec((NC, _ROWS_BLK, 1), lambda i: (0, i, 0)),
        ],
        out_specs=pl.BlockSpec((_ROWS_BLK, 128), lambda i: (i, 0)),
        out_shape=jax.ShapeDtypeStruct((N_NODES, 128), jnp.bfloat16),
    )(p, cnt)


def _mm_body(r_ref, c_ref, w1_ref, b1_ref, w2_ref, o_ref):
    dinv = _inv_counts(c_ref)
    s = (r_ref[0].astype(jnp.float32) + r_ref[1].astype(jnp.float32)) * dinv
    h = lax.dot_general(s, w1_ref[...], (((1,), (1,)), ((), ())),
                        preferred_element_type=jnp.float32)
    h = jnp.maximum(h + b1_ref[...], 0.0)
    o_ref[...] = lax.dot_general(h, w2_ref[...], (((1,), (1,)), ((), ())),
                                 preferred_element_type=jnp.float32
                                 ).astype(jnp.bfloat16)


def _tc_mm(r, cnt, W1, b1, W2):
    return pl.pallas_call(
        _mm_body,
        grid=(_GRID,),
        in_specs=[
            pl.BlockSpec((NC, _ROWS_BLK, 128), lambda i: (0, i, 0)),
            pl.BlockSpec((NC, _ROWS_BLK, 1), lambda i: (0, i, 0)),
            pl.BlockSpec((D_HID, D_IN), lambda i: (0, 0)),
            pl.BlockSpec((1, D_HID), lambda i: (0, 0)),
            pl.BlockSpec((D_OUT, D_HID), lambda i: (0, 0)),
        ],
        out_specs=pl.BlockSpec((_ROWS_BLK, D_OUT), lambda i: (i, 0)),
        out_shape=jax.ShapeDtypeStruct((N_NODES, D_OUT), jnp.bfloat16),
    )(r, cnt, W1, b1, W2)


def _final_body(r_ref, c_ref, b2_ref, o_ref):
    dinv = _inv_counts(c_ref)
    r = r_ref[0].astype(jnp.float32) + r_ref[1].astype(jnp.float32)
    o_ref[...] = r * dinv + b2_ref[...]


def _tc_final(r, cnt, b2):
    return pl.pallas_call(
        _final_body,
        grid=(_GRID,),
        in_specs=[
            pl.BlockSpec((NC, _ROWS_BLK, 128), lambda i: (0, i, 0)),
            pl.BlockSpec((NC, _ROWS_BLK, 1), lambda i: (0, i, 0)),
            pl.BlockSpec((1, D_OUT), lambda i: (0, 0)),
        ],
        out_specs=pl.BlockSpec((_ROWS_BLK, D_OUT), lambda i: (i, 0)),
        out_shape=jax.ShapeDtypeStruct((N_NODES, D_OUT), jnp.float32),
    )(r, cnt, b2)


@jax.jit
def kernel(x, edge_index, W1, b1, W2, b2):
    node_idx = edge_index[0]
    hyper_idx = edge_index[1]

    p1, dcnt, bcnt = _sc_pass_counts(x.astype(jnp.bfloat16), node_idx, hyper_idx)
    dc = dcnt[:, :, 0:1]
    bc = bcnt[:, :, 0:1]
    q1 = _tc_scale(p1, bc)
    r1 = _sc_pass(q1, hyper_idx, node_idx)
    xw2 = _tc_mm(r1, dc, W1, b1.reshape(1, D_HID), W2)
    p2 = _sc_pass(xw2, node_idx, hyper_idx)
    q2 = _tc_scale(p2, bc)
    r2 = _sc_pass(q2, hyper_idx, node_idx)
    return _tc_final(r2, dc, b2.reshape(1, D_OUT))
